# baseline (speedup 1.0000x reference)
import functools

import jax
import jax.numpy as jnp
from jax import lax
from jax.experimental import pallas as pl
from jax.experimental.pallas import tpu as pltpu
from jax.experimental.pallas import tpu_sc as plsc

NUM_ENTITIES = 1000000
NUM_RELATIONS = 1000
EMBED_DIM = 64
BATCH = 16384
MARGIN = 1.0

NW = 32
B_PER_W = BATCH // NW
CHUNK = 128
NCHUNK = B_PER_W // CHUNK
GROUPS = CHUNK // 16


def _tec_kernel(ent_hbm, rel_hbm, ih_hbm, it_hbm, ir_hbm, jh_hbm, jt_hbm,
                jr_hbm, out_hbm,
                ih_v, it_v, ir_v, jh_v, jt_v, jr_v,
                h_v, t_v, r_v, hn_v, tn_v, rn_v,
                acc_v, sem):
    wid = lax.axis_index("s") * 2 + lax.axis_index("c")

    pltpu.sync_copy(ih_hbm.at[wid], ih_v)
    pltpu.sync_copy(it_hbm.at[wid], it_v)
    pltpu.sync_copy(ir_hbm.at[wid], ir_v)
    pltpu.sync_copy(jh_hbm.at[wid], jh_v)
    pltpu.sync_copy(jt_hbm.at[wid], jt_v)
    pltpu.sync_copy(jr_hbm.at[wid], jr_v)

    zero16 = jnp.zeros((16,), jnp.float32)
    acc_v[...] = zero16

    for c in range(NCHUNK):
        d0 = pltpu.make_async_copy(ent_hbm.at[ih_v.at[c]], h_v, sem)
        d1 = pltpu.make_async_copy(ent_hbm.at[it_v.at[c]], t_v, sem)
        d2 = pltpu.make_async_copy(rel_hbm.at[ir_v.at[c]], r_v, sem)
        d3 = pltpu.make_async_copy(ent_hbm.at[jh_v.at[c]], hn_v, sem)
        d4 = pltpu.make_async_copy(ent_hbm.at[jt_v.at[c]], tn_v, sem)
        d5 = pltpu.make_async_copy(rel_hbm.at[jr_v.at[c]], rn_v, sem)
        for d in (d0, d1, d2, d3, d4, d5):
            d.start()
        for d in (d0, d1, d2, d3, d4, d5):
            d.wait()

        def group_body(g, _):
            zacc = zero16
            for j in range(16):
                row = g * 16 + j
                acc = zero16
                for k in range(4):
                    sl = pl.ds(k * 16, 16)
                    dp = jnp.abs(h_v[row, sl] + r_v[row, sl] - t_v[row, sl])
                    dn = jnp.abs(hn_v[row, sl] + rn_v[row, sl] - tn_v[row, sl])
                    acc = acc + (dp - dn)
                s = jnp.broadcast_to(jnp.sum(acc), (16,))
                zacc = zacc + jnp.maximum(s + MARGIN, 0.0)
            acc_v[...] = acc_v[...] + zacc
            return 0

        lax.fori_loop(0, GROUPS, group_body, 0)

    acc_v[...] = acc_v[...] * (1.0 / 16.0)
    pltpu.sync_copy(acc_v, out_hbm.at[wid])


@jax.jit
def _transe_loss(ent_table, rel_table, ih, it, ir, jh, jt, jr):
    mesh = plsc.VectorSubcoreMesh(core_axis_name="c", subcore_axis_name="s")
    shp = (NCHUNK, CHUNK)
    partials = pl.kernel(
        _tec_kernel,
        mesh=mesh,
        out_type=jax.ShapeDtypeStruct((NW, 16), jnp.float32),
        compiler_params=pltpu.CompilerParams(
            needs_layout_passes=False, use_tc_tiling_on_sc=False),
        scratch_types=[
            pltpu.VMEM(shp, jnp.int32),
            pltpu.VMEM(shp, jnp.int32),
            pltpu.VMEM(shp, jnp.int32),
            pltpu.VMEM(shp, jnp.int32),
            pltpu.VMEM(shp, jnp.int32),
            pltpu.VMEM(shp, jnp.int32),
            pltpu.VMEM((CHUNK, EMBED_DIM), jnp.float32),
            pltpu.VMEM((CHUNK, EMBED_DIM), jnp.float32),
            pltpu.VMEM((CHUNK, EMBED_DIM), jnp.float32),
            pltpu.VMEM((CHUNK, EMBED_DIM), jnp.float32),
            pltpu.VMEM((CHUNK, EMBED_DIM), jnp.float32),
            pltpu.VMEM((CHUNK, EMBED_DIM), jnp.float32),
            pltpu.VMEM((16,), jnp.float32),
            pltpu.SemaphoreType.DMA,
        ],
    )(ent_table, rel_table, ih, it, ir, jh, jt, jr)
    return jnp.sum(partials)


def kernel(ent_table, rel_table, pos_entities, pos_relations, neg_entities,
           neg_relations):
    shp = (NW, NCHUNK, CHUNK)
    ih = pos_entities[:, 0].reshape(shp)
    it = pos_entities[:, 1].reshape(shp)
    ir = pos_relations.reshape(shp)
    jh = neg_entities[:, 0].reshape(shp)
    jt = neg_entities[:, 1].reshape(shp)
    jr = neg_relations.reshape(shp)
    return _transe_loss(ent_table, rel_table, ih, it, ir, jh, jt, jr)
